# Initial kernel scaffold; baseline (speedup 1.0000x reference)
import jax
import jax.numpy as jnp
from jax.experimental import pallas as pl


def kernel(Attributes, cluster_labels):
    raise NotImplementedError("write your pallas kernel here")


# speedup vs baseline: 4.7767x; 4.7767x over previous
import functools

import jax
import jax.numpy as jnp
from jax import lax
from jax.experimental import pallas as pl
from jax.experimental.pallas import tpu as pltpu

_C = 64
_D = 128
_N = 320000
_RA = 2000
_RB = 2000


def _seg_kernel(x_ref, lab_ref, sums_ref, counts_ref):
    i = pl.program_id(0)

    @pl.when(i == 0)
    def _():
        sums_ref[...] = jnp.zeros_like(sums_ref)
        counts_ref[...] = jnp.zeros_like(counts_ref)

    lab = lab_ref[0, 0, :]
    clusters = lax.broadcasted_iota(jnp.int32, (_C, lab.shape[0]), 0)
    onehot_t = (clusters == lab[None, :]).astype(jnp.float32)
    sums_ref[...] += jnp.dot(onehot_t, x_ref[...],
                             preferred_element_type=jnp.float32)
    counts_ref[...] += jnp.sum(onehot_t, axis=1, keepdims=True)


def _dist_kernel(sums_ref, counts_ref, x_ref, lab_ref, out_ref,
                 cent_ref, acc_ref, nc_ref):
    i = pl.program_id(0)
    nb = pl.num_programs(0)

    @pl.when(i == 0)
    def _():
        counts = counts_ref[:, 0:1]
        nc = jnp.sum((counts > 0.0).astype(jnp.int32))
        nc_ref[0] = nc
        cent_ref[...] = sums_ref[...] / jnp.maximum(counts, 1.0)
        acc_ref[0] = 0.0

    nc = nc_ref[0]
    lab = lab_ref[0, 0, :]
    g = jnp.minimum(lab, nc - 1)
    onehot = (g[:, None] ==
              lax.broadcasted_iota(jnp.int32, (g.shape[0], _C), 1)
              ).astype(jnp.float32)
    cent_rows = jnp.dot(onehot, cent_ref[...],
                        preferred_element_type=jnp.float32)
    diff = x_ref[...] - cent_rows
    dist = jnp.sqrt(jnp.sum(diff * diff, axis=1, keepdims=True))
    acc_ref[0] += jnp.sum(dist)

    @pl.when(i == nb - 1)
    def _():
        nc_f = nc.astype(jnp.float32)
        counts = counts_ref[:, 0:1]
        gm = jnp.sum(sums_ref[...], axis=0, keepdims=True) / float(_N)
        dc = cent_ref[...] - gm
        d = jnp.sqrt(jnp.sum(dc * dc, axis=1, keepdims=True))
        cidx = lax.broadcasted_iota(jnp.int32, (_C, 1), 0)
        between = jnp.sum(jnp.where(cidx < nc, counts * d, 0.0)) / (nc_f - 1.0)
        within = acc_ref[0] / (float(_N) - nc_f)
        out_ref[...] = jnp.full((1, 1), between / within, dtype=jnp.float32)


def _segment_sums(x, labels):
    nba = _N // _RA
    lab3 = labels.reshape(nba, 1, _RA)
    return pl.pallas_call(
        _seg_kernel,
        grid=(nba,),
        in_specs=[
            pl.BlockSpec((_RA, _D), lambda i: (i, 0)),
            pl.BlockSpec((1, 1, _RA), lambda i: (i, 0, 0)),
        ],
        out_specs=[
            pl.BlockSpec((_C, _D), lambda i: (0, 0)),
            pl.BlockSpec((_C, 1), lambda i: (0, 0)),
        ],
        out_shape=[
            jax.ShapeDtypeStruct((_C, _D), jnp.float32),
            jax.ShapeDtypeStruct((_C, 1), jnp.float32),
        ],
    )(x, lab3)


def _distance_pass(sums, counts, x, labels):
    nbb = _N // _RB
    lab3 = labels.reshape(nbb, 1, _RB)
    return pl.pallas_call(
        _dist_kernel,
        grid=(nbb,),
        in_specs=[
            pl.BlockSpec((_C, _D), lambda i: (0, 0)),
            pl.BlockSpec((_C, 1), lambda i: (0, 0)),
            pl.BlockSpec((_RB, _D), lambda i: (i, 0)),
            pl.BlockSpec((1, 1, _RB), lambda i: (i, 0, 0)),
        ],
        out_specs=pl.BlockSpec((1, 1), lambda i: (0, 0)),
        out_shape=jax.ShapeDtypeStruct((1, 1), jnp.float32),
        scratch_shapes=[
            pltpu.VMEM((_C, _D), jnp.float32),
            pltpu.SMEM((1,), jnp.float32),
            pltpu.SMEM((1,), jnp.int32),
        ],
    )(sums, counts, x, lab3)


def kernel(Attributes, cluster_labels):
    labels = cluster_labels[0]
    sums, counts = _segment_sums(Attributes, labels)
    loss = _distance_pass(sums, counts, Attributes, labels)
    return loss.reshape(1)


# speedup vs baseline: 10.0658x; 2.1073x over previous
import functools

import jax
import jax.numpy as jnp
from jax import lax
from jax.experimental import pallas as pl
from jax.experimental.pallas import tpu as pltpu

_C = 64
_D = 128
_N = 320000
_RA = 8000
_RB = 8000


def _seg_kernel(x_ref, lab_ref, sums_ref, counts_ref):
    i = pl.program_id(0)

    @pl.when(i == 0)
    def _():
        sums_ref[...] = jnp.zeros_like(sums_ref)
        counts_ref[...] = jnp.zeros_like(counts_ref)

    lab = lab_ref[0, 0, :]
    clusters = lax.broadcasted_iota(jnp.int32, (_C, lab.shape[0]), 0)
    onehot_t = (clusters == lab[None, :]).astype(jnp.float32)
    sums_ref[...] += jnp.dot(onehot_t, x_ref[...],
                             preferred_element_type=jnp.float32)
    counts_ref[...] += jnp.sum(onehot_t, axis=1, keepdims=True)


def _dist_kernel(sums_ref, counts_ref, x_ref, lab_ref, out_ref,
                 cent_ref, acc_ref, nc_ref):
    i = pl.program_id(0)
    nb = pl.num_programs(0)

    @pl.when(i == 0)
    def _():
        counts = counts_ref[:, 0:1]
        nc = jnp.sum((counts > 0.0).astype(jnp.int32))
        nc_ref[0] = nc
        cent_ref[...] = sums_ref[...] / jnp.maximum(counts, 1.0)
        acc_ref[0] = 0.0

    nc = nc_ref[0]
    lab = lab_ref[0, 0, :]
    g = jnp.minimum(lab, nc - 1)
    onehot = (g[:, None] ==
              lax.broadcasted_iota(jnp.int32, (g.shape[0], _C), 1)
              ).astype(jnp.float32)
    cent_rows = jnp.dot(onehot, cent_ref[...],
                        preferred_element_type=jnp.float32)
    diff = x_ref[...] - cent_rows
    dist = jnp.sqrt(jnp.sum(diff * diff, axis=1, keepdims=True))
    acc_ref[0] += jnp.sum(dist)

    @pl.when(i == nb - 1)
    def _():
        nc_f = nc.astype(jnp.float32)
        counts = counts_ref[:, 0:1]
        gm = jnp.sum(sums_ref[...], axis=0, keepdims=True) / float(_N)
        dc = cent_ref[...] - gm
        d = jnp.sqrt(jnp.sum(dc * dc, axis=1, keepdims=True))
        cidx = lax.broadcasted_iota(jnp.int32, (_C, 1), 0)
        between = jnp.sum(jnp.where(cidx < nc, counts * d, 0.0)) / (nc_f - 1.0)
        within = acc_ref[0] / (float(_N) - nc_f)
        out_ref[...] = jnp.full((1, 1), between / within, dtype=jnp.float32)


def _segment_sums(x, labels):
    nba = _N // _RA
    lab3 = labels.reshape(nba, 1, _RA)
    return pl.pallas_call(
        _seg_kernel,
        grid=(nba,),
        in_specs=[
            pl.BlockSpec((_RA, _D), lambda i: (i, 0)),
            pl.BlockSpec((1, 1, _RA), lambda i: (i, 0, 0)),
        ],
        out_specs=[
            pl.BlockSpec((_C, _D), lambda i: (0, 0)),
            pl.BlockSpec((_C, 1), lambda i: (0, 0)),
        ],
        out_shape=[
            jax.ShapeDtypeStruct((_C, _D), jnp.float32),
            jax.ShapeDtypeStruct((_C, 1), jnp.float32),
        ],
    )(x, lab3)


def _distance_pass(sums, counts, x, labels):
    nbb = _N // _RB
    lab3 = labels.reshape(nbb, 1, _RB)
    return pl.pallas_call(
        _dist_kernel,
        grid=(nbb,),
        in_specs=[
            pl.BlockSpec((_C, _D), lambda i: (0, 0)),
            pl.BlockSpec((_C, 1), lambda i: (0, 0)),
            pl.BlockSpec((_RB, _D), lambda i: (i, 0)),
            pl.BlockSpec((1, 1, _RB), lambda i: (i, 0, 0)),
        ],
        out_specs=pl.BlockSpec((1, 1), lambda i: (0, 0)),
        out_shape=jax.ShapeDtypeStruct((1, 1), jnp.float32),
        scratch_shapes=[
            pltpu.VMEM((_C, _D), jnp.float32),
            pltpu.SMEM((1,), jnp.float32),
            pltpu.SMEM((1,), jnp.int32),
        ],
    )(sums, counts, x, lab3)


def kernel(Attributes, cluster_labels):
    labels = cluster_labels[0]
    sums, counts = _segment_sums(Attributes, labels)
    loss = _distance_pass(sums, counts, Attributes, labels)
    return loss.reshape(1)


# speedup vs baseline: 13.0001x; 1.2915x over previous
import functools

import jax
import jax.numpy as jnp
from jax import lax
from jax.experimental import pallas as pl
from jax.experimental.pallas import tpu as pltpu

_C = 64
_D = 128
_N = 320000
_RA = 16000
_RB = 16000


def _seg_kernel(x_ref, lab_ref, sums_ref, counts_ref):
    i = pl.program_id(0)

    @pl.when(i == 0)
    def _():
        sums_ref[...] = jnp.zeros_like(sums_ref)
        counts_ref[...] = jnp.zeros_like(counts_ref)

    lab = lab_ref[0, 0, :]
    clusters = lax.broadcasted_iota(jnp.int32, (_C, lab.shape[0]), 0)
    onehot_t = (clusters == lab[None, :]).astype(jnp.float32)
    sums_ref[...] += jnp.dot(onehot_t, x_ref[...],
                             preferred_element_type=jnp.float32)
    counts_ref[...] += jnp.sum(onehot_t, axis=1, keepdims=True)


def _dist_kernel(sums_ref, counts_ref, x_ref, lab_ref, out_ref,
                 cent_ref, acc_ref, nc_ref):
    i = pl.program_id(0)
    nb = pl.num_programs(0)

    @pl.when(i == 0)
    def _():
        counts = counts_ref[:, 0:1]
        nc = jnp.sum((counts > 0.0).astype(jnp.int32))
        nc_ref[0] = nc
        cent_ref[...] = sums_ref[...] / jnp.maximum(counts, 1.0)
        acc_ref[0] = 0.0

    nc = nc_ref[0]
    lab = lab_ref[0, 0, :]
    g = jnp.minimum(lab, nc - 1)
    onehot = (g[:, None] ==
              lax.broadcasted_iota(jnp.int32, (g.shape[0], _C), 1)
              ).astype(jnp.float32)
    cent_rows = jnp.dot(onehot, cent_ref[...],
                        preferred_element_type=jnp.float32)
    diff = x_ref[...] - cent_rows
    dist = jnp.sqrt(jnp.sum(diff * diff, axis=1, keepdims=True))
    acc_ref[0] += jnp.sum(dist)

    @pl.when(i == nb - 1)
    def _():
        nc_f = nc.astype(jnp.float32)
        counts = counts_ref[:, 0:1]
        gm = jnp.sum(sums_ref[...], axis=0, keepdims=True) / float(_N)
        dc = cent_ref[...] - gm
        d = jnp.sqrt(jnp.sum(dc * dc, axis=1, keepdims=True))
        cidx = lax.broadcasted_iota(jnp.int32, (_C, 1), 0)
        between = jnp.sum(jnp.where(cidx < nc, counts * d, 0.0)) / (nc_f - 1.0)
        within = acc_ref[0] / (float(_N) - nc_f)
        out_ref[...] = jnp.full((1, 1), between / within, dtype=jnp.float32)


def _segment_sums(x, labels):
    nba = _N // _RA
    lab3 = labels.reshape(nba, 1, _RA)
    return pl.pallas_call(
        _seg_kernel,
        grid=(nba,),
        in_specs=[
            pl.BlockSpec((_RA, _D), lambda i: (i, 0)),
            pl.BlockSpec((1, 1, _RA), lambda i: (i, 0, 0)),
        ],
        out_specs=[
            pl.BlockSpec((_C, _D), lambda i: (0, 0)),
            pl.BlockSpec((_C, 1), lambda i: (0, 0)),
        ],
        out_shape=[
            jax.ShapeDtypeStruct((_C, _D), jnp.float32),
            jax.ShapeDtypeStruct((_C, 1), jnp.float32),
        ],
    )(x, lab3)


def _distance_pass(sums, counts, x, labels):
    nbb = _N // _RB
    lab3 = labels.reshape(nbb, 1, _RB)
    return pl.pallas_call(
        _dist_kernel,
        grid=(nbb,),
        in_specs=[
            pl.BlockSpec((_C, _D), lambda i: (0, 0)),
            pl.BlockSpec((_C, 1), lambda i: (0, 0)),
            pl.BlockSpec((_RB, _D), lambda i: (i, 0)),
            pl.BlockSpec((1, 1, _RB), lambda i: (i, 0, 0)),
        ],
        out_specs=pl.BlockSpec((1, 1), lambda i: (0, 0)),
        out_shape=jax.ShapeDtypeStruct((1, 1), jnp.float32),
        scratch_shapes=[
            pltpu.VMEM((_C, _D), jnp.float32),
            pltpu.SMEM((1,), jnp.float32),
            pltpu.SMEM((1,), jnp.int32),
        ],
    )(sums, counts, x, lab3)


def kernel(Attributes, cluster_labels):
    labels = cluster_labels[0]
    sums, counts = _segment_sums(Attributes, labels)
    loss = _distance_pass(sums, counts, Attributes, labels)
    return loss.reshape(1)


# speedup vs baseline: 13.1720x; 1.0132x over previous
import functools

import jax
import jax.numpy as jnp
from jax import lax
from jax.experimental import pallas as pl
from jax.experimental.pallas import tpu as pltpu

_C = 64
_D = 128
_N = 320000
_RA = 32000
_RB = 32000


def _seg_kernel(x_ref, lab_ref, sums_ref, counts_ref):
    i = pl.program_id(0)

    @pl.when(i == 0)
    def _():
        sums_ref[...] = jnp.zeros_like(sums_ref)
        counts_ref[...] = jnp.zeros_like(counts_ref)

    lab = lab_ref[0, 0, :]
    clusters = lax.broadcasted_iota(jnp.int32, (_C, lab.shape[0]), 0)
    onehot_t = (clusters == lab[None, :]).astype(jnp.float32)
    sums_ref[...] += jnp.dot(onehot_t, x_ref[...],
                             preferred_element_type=jnp.float32)
    counts_ref[...] += jnp.sum(onehot_t, axis=1, keepdims=True)


def _dist_kernel(sums_ref, counts_ref, x_ref, lab_ref, out_ref,
                 cent_ref, acc_ref, nc_ref):
    i = pl.program_id(0)
    nb = pl.num_programs(0)

    @pl.when(i == 0)
    def _():
        counts = counts_ref[:, 0:1]
        nc = jnp.sum((counts > 0.0).astype(jnp.int32))
        nc_ref[0] = nc
        cent_ref[...] = sums_ref[...] / jnp.maximum(counts, 1.0)
        acc_ref[0] = 0.0

    nc = nc_ref[0]
    lab = lab_ref[0, 0, :]
    g = jnp.minimum(lab, nc - 1)
    onehot = (g[:, None] ==
              lax.broadcasted_iota(jnp.int32, (g.shape[0], _C), 1)
              ).astype(jnp.float32)
    cent_rows = jnp.dot(onehot, cent_ref[...],
                        preferred_element_type=jnp.float32)
    diff = x_ref[...] - cent_rows
    dist = jnp.sqrt(jnp.sum(diff * diff, axis=1, keepdims=True))
    acc_ref[0] += jnp.sum(dist)

    @pl.when(i == nb - 1)
    def _():
        nc_f = nc.astype(jnp.float32)
        counts = counts_ref[:, 0:1]
        gm = jnp.sum(sums_ref[...], axis=0, keepdims=True) / float(_N)
        dc = cent_ref[...] - gm
        d = jnp.sqrt(jnp.sum(dc * dc, axis=1, keepdims=True))
        cidx = lax.broadcasted_iota(jnp.int32, (_C, 1), 0)
        between = jnp.sum(jnp.where(cidx < nc, counts * d, 0.0)) / (nc_f - 1.0)
        within = acc_ref[0] / (float(_N) - nc_f)
        out_ref[...] = jnp.full((1, 1), between / within, dtype=jnp.float32)


def _segment_sums(x, labels):
    nba = _N // _RA
    lab3 = labels.reshape(nba, 1, _RA)
    return pl.pallas_call(
        _seg_kernel,
        grid=(nba,),
        in_specs=[
            pl.BlockSpec((_RA, _D), lambda i: (i, 0)),
            pl.BlockSpec((1, 1, _RA), lambda i: (i, 0, 0)),
        ],
        out_specs=[
            pl.BlockSpec((_C, _D), lambda i: (0, 0)),
            pl.BlockSpec((_C, 1), lambda i: (0, 0)),
        ],
        out_shape=[
            jax.ShapeDtypeStruct((_C, _D), jnp.float32),
            jax.ShapeDtypeStruct((_C, 1), jnp.float32),
        ],
    )(x, lab3)


def _distance_pass(sums, counts, x, labels):
    nbb = _N // _RB
    lab3 = labels.reshape(nbb, 1, _RB)
    return pl.pallas_call(
        _dist_kernel,
        grid=(nbb,),
        in_specs=[
            pl.BlockSpec((_C, _D), lambda i: (0, 0)),
            pl.BlockSpec((_C, 1), lambda i: (0, 0)),
            pl.BlockSpec((_RB, _D), lambda i: (i, 0)),
            pl.BlockSpec((1, 1, _RB), lambda i: (i, 0, 0)),
        ],
        out_specs=pl.BlockSpec((1, 1), lambda i: (0, 0)),
        out_shape=jax.ShapeDtypeStruct((1, 1), jnp.float32),
        scratch_shapes=[
            pltpu.VMEM((_C, _D), jnp.float32),
            pltpu.SMEM((1,), jnp.float32),
            pltpu.SMEM((1,), jnp.int32),
        ],
    )(sums, counts, x, lab3)


def kernel(Attributes, cluster_labels):
    labels = cluster_labels[0]
    sums, counts = _segment_sums(Attributes, labels)
    loss = _distance_pass(sums, counts, Attributes, labels)
    return loss.reshape(1)


# speedup vs baseline: 13.7025x; 1.0403x over previous
import functools

import jax
import jax.numpy as jnp
from jax import lax
from jax.experimental import pallas as pl
from jax.experimental.pallas import tpu as pltpu

_C = 64
_D = 128
_N = 320000
_RA = 32000
_RB = 32000


def _seg_kernel(x_ref, lab_ref, sums_ref, counts_ref):
    i = pl.program_id(0)

    @pl.when(i == 0)
    def _():
        sums_ref[...] = jnp.zeros_like(sums_ref)
        counts_ref[...] = jnp.zeros_like(counts_ref)

    lab = lab_ref[0, 0, :]
    clusters = lax.broadcasted_iota(jnp.int32, (_C, lab.shape[0]), 0)
    onehot_t = (clusters == lab[None, :]).astype(jnp.float32)
    sums_ref[...] += jnp.dot(onehot_t, x_ref[...],
                             preferred_element_type=jnp.float32)
    counts_ref[...] += jnp.sum(onehot_t, axis=1, keepdims=True)


def _dist_kernel(sums_ref, counts_ref, x_ref, lab_ref, out_ref,
                 cent_ref, acc_ref, nc_ref):
    i = pl.program_id(0)
    nb = pl.num_programs(0)

    @pl.when(i == 0)
    def _():
        counts = counts_ref[:, 0:1]
        nc = jnp.sum((counts > 0.0).astype(jnp.int32))
        nc_ref[0] = nc
        cent_ref[...] = sums_ref[...] / jnp.maximum(counts, 1.0)
        acc_ref[0] = 0.0

    nc = nc_ref[0]
    lab = lab_ref[0, 0, :]
    g = jnp.minimum(lab, nc - 1)
    onehot = (g[:, None] ==
              lax.broadcasted_iota(jnp.int32, (g.shape[0], _C), 1)
              ).astype(jnp.float32)
    cent_rows = jnp.dot(onehot, cent_ref[...],
                        preferred_element_type=jnp.float32)
    diff = x_ref[...] - cent_rows
    ones_mat = jnp.ones((_D, _D), dtype=jnp.float32)
    e_dup = jnp.dot(diff * diff, ones_mat,
                    preferred_element_type=jnp.float32)
    acc_ref[0] += jnp.sum(jnp.sqrt(e_dup)) * (1.0 / float(_D))

    @pl.when(i == nb - 1)
    def _():
        nc_f = nc.astype(jnp.float32)
        counts = counts_ref[:, 0:1]
        gm = jnp.sum(sums_ref[...], axis=0, keepdims=True) / float(_N)
        dc = cent_ref[...] - gm
        d = jnp.sqrt(jnp.sum(dc * dc, axis=1, keepdims=True))
        cidx = lax.broadcasted_iota(jnp.int32, (_C, 1), 0)
        between = jnp.sum(jnp.where(cidx < nc, counts * d, 0.0)) / (nc_f - 1.0)
        within = acc_ref[0] / (float(_N) - nc_f)
        out_ref[...] = jnp.full((1, 1), between / within, dtype=jnp.float32)


def _segment_sums(x, labels):
    nba = _N // _RA
    lab3 = labels.reshape(nba, 1, _RA)
    return pl.pallas_call(
        _seg_kernel,
        grid=(nba,),
        in_specs=[
            pl.BlockSpec((_RA, _D), lambda i: (i, 0)),
            pl.BlockSpec((1, 1, _RA), lambda i: (i, 0, 0)),
        ],
        out_specs=[
            pl.BlockSpec((_C, _D), lambda i: (0, 0)),
            pl.BlockSpec((_C, 1), lambda i: (0, 0)),
        ],
        out_shape=[
            jax.ShapeDtypeStruct((_C, _D), jnp.float32),
            jax.ShapeDtypeStruct((_C, 1), jnp.float32),
        ],
    )(x, lab3)


def _distance_pass(sums, counts, x, labels):
    nbb = _N // _RB
    lab3 = labels.reshape(nbb, 1, _RB)
    return pl.pallas_call(
        _dist_kernel,
        grid=(nbb,),
        in_specs=[
            pl.BlockSpec((_C, _D), lambda i: (0, 0)),
            pl.BlockSpec((_C, 1), lambda i: (0, 0)),
            pl.BlockSpec((_RB, _D), lambda i: (i, 0)),
            pl.BlockSpec((1, 1, _RB), lambda i: (i, 0, 0)),
        ],
        out_specs=pl.BlockSpec((1, 1), lambda i: (0, 0)),
        out_shape=jax.ShapeDtypeStruct((1, 1), jnp.float32),
        scratch_shapes=[
            pltpu.VMEM((_C, _D), jnp.float32),
            pltpu.SMEM((1,), jnp.float32),
            pltpu.SMEM((1,), jnp.int32),
        ],
    )(sums, counts, x, lab3)


def kernel(Attributes, cluster_labels):
    labels = cluster_labels[0]
    sums, counts = _segment_sums(Attributes, labels)
    loss = _distance_pass(sums, counts, Attributes, labels)
    return loss.reshape(1)


# speedup vs baseline: 15.3260x; 1.1185x over previous
import jax
import jax.numpy as jnp
from jax import lax
from jax.experimental import pallas as pl
from jax.experimental.pallas import tpu as pltpu

_C = 64
_D = 128
_N = 320000
_R = 32000


def _fused_kernel(x_ref, lab_ref, out_ref,
                  sums_ref, counts_ref, cent_ref, acc_ref, nc_ref):
    p = pl.program_id(0)
    i = pl.program_id(1)
    nb = pl.num_programs(1)

    @pl.when((p == 0) & (i == 0))
    def _():
        sums_ref[...] = jnp.zeros_like(sums_ref)
        counts_ref[...] = jnp.zeros_like(counts_ref)

    @pl.when(p == 0)
    def _():
        lab = lab_ref[0, 0, :]
        clusters = lax.broadcasted_iota(jnp.int32, (_C, _R), 0)
        onehot_t = (clusters == lab[None, :]).astype(jnp.float32)
        sums_ref[...] += jnp.dot(onehot_t, x_ref[...],
                                 preferred_element_type=jnp.float32)
        counts_ref[...] += jnp.sum(onehot_t, axis=1, keepdims=True)

    @pl.when((p == 1) & (i == 0))
    def _():
        counts = counts_ref[...]
        nc_ref[0] = jnp.sum((counts > 0.0).astype(jnp.int32))
        cent_ref[...] = sums_ref[...] / jnp.maximum(counts, 1.0)
        acc_ref[0] = 0.0

    @pl.when(p == 1)
    def _():
        nc = nc_ref[0]
        lab = lab_ref[0, 0, :]
        g = jnp.minimum(lab, nc - 1)
        onehot = (g[:, None] ==
                  lax.broadcasted_iota(jnp.int32, (_R, _C), 1)
                  ).astype(jnp.float32)
        cent_rows = jnp.dot(onehot, cent_ref[...],
                            preferred_element_type=jnp.float32)
        diff = x_ref[...] - cent_rows
        ones_mat = jnp.ones((_D, _D), dtype=jnp.float32)
        e_dup = jnp.dot(diff * diff, ones_mat,
                        preferred_element_type=jnp.float32)
        dist = e_dup * lax.rsqrt(e_dup + 1e-37)
        acc_ref[0] += jnp.sum(dist) * (1.0 / float(_D))

        @pl.when(i == nb - 1)
        def _():
            nc_f = nc.astype(jnp.float32)
            counts = counts_ref[...]
            gm = jnp.sum(sums_ref[...], axis=0, keepdims=True) / float(_N)
            dc = cent_ref[...] - gm
            e_c = jnp.sum(dc * dc, axis=1, keepdims=True)
            d = e_c * lax.rsqrt(e_c + 1e-37)
            cidx = lax.broadcasted_iota(jnp.int32, (_C, 1), 0)
            between = (jnp.sum(jnp.where(cidx < nc, counts * d, 0.0))
                       / (nc_f - 1.0))
            within = acc_ref[0] / (float(_N) - nc_f)
            out_ref[...] = jnp.full((1, 1), between / within,
                                    dtype=jnp.float32)


def kernel(Attributes, cluster_labels):
    nb = _N // _R
    lab3 = cluster_labels.reshape(nb, 1, _R)
    loss = pl.pallas_call(
        _fused_kernel,
        grid=(2, nb),
        in_specs=[
            pl.BlockSpec((_R, _D), lambda p, i: (i, 0)),
            pl.BlockSpec((1, 1, _R), lambda p, i: (i, 0, 0)),
        ],
        out_specs=pl.BlockSpec((1, 1), lambda p, i: (0, 0)),
        out_shape=jax.ShapeDtypeStruct((1, 1), jnp.float32),
        scratch_shapes=[
            pltpu.VMEM((_C, _D), jnp.float32),
            pltpu.VMEM((_C, 1), jnp.float32),
            pltpu.VMEM((_C, _D), jnp.float32),
            pltpu.SMEM((1,), jnp.float32),
            pltpu.SMEM((1,), jnp.int32),
        ],
    )(Attributes, lab3)
    return loss.reshape(1)
